# Initial kernel scaffold; baseline (speedup 1.0000x reference)
import jax
import jax.numpy as jnp
from jax.experimental import pallas as pl


def kernel(charge, numbers, mol_idx, emb, W1, b1, w2, sae_tensor):
    raise NotImplementedError("write your pallas kernel here")


# speedup vs baseline: 16.4770x; 16.4770x over previous
import functools

import jax
import jax.numpy as jnp
from jax import lax
from jax.experimental import pallas as pl
from jax.experimental.pallas import tpu as pltpu
from jax.experimental.pallas import tpu_sc as plsc

_FID_SHIFT = 200
_NUM_MOLS = 16
_ZPAD = 128
_LANES = 16
_NSUB = 16


def _table_body(emb_ref, w1_ref, b1_ref, w2_ref, sae_ref, out_ref):
    ht = lax.dot_general(w1_ref[...], emb_ref[...], (((0,), (1,)), ((), ())),
                         preferred_element_type=jnp.float32)
    ht = jnp.maximum(ht + b1_ref[...], 0.0)
    t = lax.dot_general(w2_ref[...], ht, (((1,), (0,)), ((), ())),
                        preferred_element_type=jnp.float32)
    out_ref[...] = t + sae_ref[...]


@functools.lru_cache(maxsize=None)
def _make_sc_segment(n_atoms: int):
    chunk = n_atoms // _NSUB
    steps = chunk // _LANES
    assert chunk * _NSUB == n_atoms and steps * _LANES == chunk

    mesh = plsc.VectorSubcoreMesh(core_axis_name="c", subcore_axis_name="s",
                                  num_cores=1)

    @functools.partial(
        pl.kernel,
        out_type=jax.ShapeDtypeStruct((_NUM_MOLS,), jnp.float32),
        mesh=mesh,
        compiler_params=pltpu.CompilerParams(needs_layout_passes=False),
        scratch_types=[
            pltpu.VMEM((chunk,), jnp.int32),
            pltpu.VMEM((chunk,), jnp.int32),
            pltpu.VMEM((_ZPAD,), jnp.float32),
            pltpu.VMEM((_LANES * _NUM_MOLS,), jnp.float32),
            pltpu.VMEM((_NUM_MOLS,), jnp.float32),
            pltpu.VMEM((_NSUB * _NUM_MOLS,), jnp.float32),
            pltpu.VMEM_SHARED((_NSUB * _NUM_MOLS,), jnp.float32),
        ],
    )
    def sc_segment(z_hbm, mol_hbm, tab_hbm, out_hbm,
                   z_v, mol_v, tab_v, acc_v, part_v, all_v, shared):
        sid = lax.axis_index("s")
        base = sid * chunk
        pltpu.sync_copy(z_hbm.at[pl.ds(base, chunk)], z_v)
        pltpu.sync_copy(mol_hbm.at[pl.ds(base, chunk)], mol_v)
        pltpu.sync_copy(tab_hbm, tab_v)

        lane = lax.iota(jnp.int32, _LANES)
        for i in range(_NUM_MOLS):
            acc_v[pl.ds(i * _LANES, _LANES)] = jnp.zeros((_LANES,), jnp.float32)

        def step(i, carry):
            z = z_v[pl.ds(i * _LANES, _LANES)]
            m = mol_v[pl.ds(i * _LANES, _LANES)]
            v = plsc.load_gather(tab_v, [z])
            plsc.addupdate_scatter(acc_v, [lane * _NUM_MOLS + m], v)
            return carry

        lax.fori_loop(0, steps, step, 0)

        r = jnp.zeros((_NUM_MOLS,), jnp.float32)
        for l in range(_LANES):
            r = r + acc_v[pl.ds(l * _NUM_MOLS, _NUM_MOLS)]
        part_v[...] = r
        pltpu.sync_copy(part_v, shared.at[pl.ds(sid * _NUM_MOLS, _NUM_MOLS)])
        plsc.subcore_barrier()

        @pl.when(sid == 0)
        def _():
            pltpu.sync_copy(shared, all_v)
            total = jnp.zeros((_NUM_MOLS,), jnp.float32)
            for w in range(_NSUB):
                total = total + all_v[pl.ds(w * _NUM_MOLS, _NUM_MOLS)]
            part_v[...] = total
            pltpu.sync_copy(part_v, out_hbm)

    return sc_segment


def kernel(charge, numbers, mol_idx, emb, W1, b1, w2, sae_tensor):
    del charge
    n_atoms = numbers.shape[0]
    emb_pad = jnp.zeros((_ZPAD, emb.shape[1]), emb.dtype).at[:emb.shape[0]].set(emb)
    sae_row = lax.dynamic_slice(sae_tensor, (_FID_SHIFT,), (_ZPAD,)).reshape(1, _ZPAD)

    table = pl.pallas_call(
        _table_body,
        out_shape=jax.ShapeDtypeStruct((1, _ZPAD), jnp.float32),
    )(emb_pad, W1, b1.reshape(-1, 1), w2.reshape(1, -1), sae_row)

    return _make_sc_segment(n_atoms)(numbers, mol_idx, table.reshape(_ZPAD))


# speedup vs baseline: 19.1929x; 1.1648x over previous
import functools

import jax
import jax.numpy as jnp
from jax import lax
from jax.experimental import pallas as pl
from jax.experimental.pallas import tpu as pltpu
from jax.experimental.pallas import tpu_sc as plsc

_FID_SHIFT = 200
_NUM_MOLS = 16
_MAXZ = 120
_LANES = 16
_NSUB = 16


def _table_body(emb_ref, w1_ref, b1_ref, w2_ref, sae_ref, out_ref):
    ht = lax.dot_general(w1_ref[...], emb_ref[...], (((0,), (1,)), ((), ())),
                         preferred_element_type=jnp.float32)
    ht = jnp.maximum(ht + b1_ref[...].reshape(-1, 1), 0.0)
    t = lax.dot_general(w2_ref[...].reshape(1, -1), ht, (((1,), (0,)), ((), ())),
                        preferred_element_type=jnp.float32)
    out_ref[...] = (t + sae_ref[pl.ds(_FID_SHIFT, _MAXZ)].reshape(1, -1))[0]


@functools.lru_cache(maxsize=None)
def _make_sc_segment(n_atoms: int):
    chunk = n_atoms // _NSUB
    steps = chunk // _LANES
    assert chunk * _NSUB == n_atoms and steps * _LANES == chunk

    mesh = plsc.VectorSubcoreMesh(core_axis_name="c", subcore_axis_name="s",
                                  num_cores=1)

    @functools.partial(
        pl.kernel,
        out_type=jax.ShapeDtypeStruct((_NUM_MOLS,), jnp.float32),
        mesh=mesh,
        compiler_params=pltpu.CompilerParams(needs_layout_passes=False),
        scratch_types=[
            pltpu.VMEM((chunk,), jnp.int32),
            pltpu.VMEM((chunk,), jnp.int32),
            pltpu.VMEM((_MAXZ,), jnp.float32),
            pltpu.VMEM((_LANES * _NUM_MOLS,), jnp.float32),
            pltpu.VMEM((_NUM_MOLS,), jnp.float32),
            pltpu.VMEM((_NSUB * _NUM_MOLS,), jnp.float32),
            pltpu.VMEM_SHARED((_NSUB * _NUM_MOLS,), jnp.float32),
        ],
    )
    def sc_segment(z_hbm, mol_hbm, tab_hbm, out_hbm,
                   z_v, mol_v, tab_v, acc_v, part_v, all_v, shared):
        sid = lax.axis_index("s")
        base = sid * chunk
        pltpu.sync_copy(z_hbm.at[pl.ds(base, chunk)], z_v)
        pltpu.sync_copy(mol_hbm.at[pl.ds(base, chunk)], mol_v)
        pltpu.sync_copy(tab_hbm, tab_v)

        lane16 = lax.iota(jnp.int32, _LANES) * _NUM_MOLS
        for i in range(_NUM_MOLS):
            acc_v[pl.ds(i * _LANES, _LANES)] = jnp.zeros((_LANES,), jnp.float32)

        @plsc.parallel_loop(0, steps, 1, unroll=8)
        def _step(i):
            z = z_v[pl.ds(i * _LANES, _LANES)]
            m = mol_v[pl.ds(i * _LANES, _LANES)]
            v = plsc.load_gather(tab_v, [z])
            plsc.addupdate_scatter(acc_v, [lane16 + m], v)

        r = jnp.zeros((_NUM_MOLS,), jnp.float32)
        for l in range(_LANES):
            r = r + acc_v[pl.ds(l * _NUM_MOLS, _NUM_MOLS)]
        part_v[...] = r
        pltpu.sync_copy(part_v, shared.at[pl.ds(sid * _NUM_MOLS, _NUM_MOLS)])
        plsc.subcore_barrier()

        @pl.when(sid == 0)
        def _():
            pltpu.sync_copy(shared, all_v)
            total = jnp.zeros((_NUM_MOLS,), jnp.float32)
            for w in range(_NSUB):
                total = total + all_v[pl.ds(w * _NUM_MOLS, _NUM_MOLS)]
            part_v[...] = total
            pltpu.sync_copy(part_v, out_hbm)

    return sc_segment


def kernel(charge, numbers, mol_idx, emb, W1, b1, w2, sae_tensor):
    del charge
    table = pl.pallas_call(
        _table_body,
        out_shape=jax.ShapeDtypeStruct((_MAXZ,), jnp.float32),
    )(emb, W1, b1, w2, sae_tensor)
    return _make_sc_segment(numbers.shape[0])(numbers, mol_idx, table)
